# Initial kernel scaffold; baseline (speedup 1.0000x reference)
import jax
import jax.numpy as jnp
from jax.experimental import pallas as pl


def kernel(features, rois):
    raise NotImplementedError("write your pallas kernel here")


# speedup vs baseline: 2.9817x; 2.9817x over previous
import functools

import jax
import jax.numpy as jnp
from jax import lax
from jax.experimental import pallas as pl
from jax.experimental.pallas import tpu as pltpu
from jax.experimental.pallas import tpu_sc as plsc

G = 7
D = 21
DP = 32
H = W = 64
HP = H + 1
SCALE = 0.0625
NB = G * G
NG = 2 * NB
NROI = 5000
NW = 32
LPG = 16
GROUPS_PER_W = 10
NPAD = NW * GROUPS_PER_W * LPG
NIDX = NB * 4 * LPG
NCHUNK = 25


def _integral_body(x_ref, o_ref):
    x = x_ref[0]
    r = lax.broadcasted_iota(jnp.int32, (H, H), 0)
    c = lax.broadcasted_iota(jnp.int32, (H, H), 1)
    upper = (r <= c).astype(jnp.float32)
    lower = (c <= r).astype(jnp.float32)
    y = lax.dot_general(x.reshape(D * H, W), upper, (((1,), (0,)), ((), ())),
                        precision=lax.Precision.HIGHEST,
                        preferred_element_type=jnp.float32)
    y = y.reshape(D, H, W)
    z = lax.dot_general(lower, y, (((1,), (1,)), ((), ())),
                        precision=lax.Precision.HIGHEST,
                        preferred_element_type=jnp.float32)
    o_ref[0] = z


def _integral(feats2):
    return pl.pallas_call(
        _integral_body,
        grid=(NG,),
        in_specs=[pl.BlockSpec((1, D, H, W), lambda i: (i, 0, 0, 0))],
        out_specs=pl.BlockSpec((1, H, D, W), lambda i: (i, 0, 0, 0)),
        out_shape=jax.ShapeDtypeStruct((NG, H, D, W), jnp.float32),
    )(feats2)


_mesh = plsc.VectorSubcoreMesh(core_axis_name="c", subcore_axis_name="s")


@functools.partial(
    pl.kernel,
    mesh=_mesh,
    compiler_params=pltpu.CompilerParams(use_tc_tiling_on_sc=False),
    out_type=jax.ShapeDtypeStruct((NPAD, NB, DP), jnp.float32),
    scratch_types=[
        pltpu.VMEM((LPG,), jnp.float32),
        pltpu.VMEM((LPG,), jnp.float32),
        pltpu.VMEM((LPG,), jnp.float32),
        pltpu.VMEM((LPG,), jnp.float32),
        pltpu.VMEM((LPG,), jnp.float32),
        pltpu.VMEM((NCHUNK, 128), jnp.int32),
        pltpu.VMEM((NCHUNK * 128, DP), jnp.float32),
        pltpu.VMEM((NB, LPG), jnp.float32),
        pltpu.VMEM((NB, DP), jnp.float32),
        pltpu.SemaphoreType.DMA,
    ],
)
def _sc_pool(table, rb, rx1, ry1, rx2, ry2, out,
             cb0, cb1, cb2, cb3, cb4, idx_v, rows_v, fac_v, outb, sem):
    wid = lax.axis_index("s") * 2 + lax.axis_index("c")

    zeros16 = jnp.zeros((16,), jnp.int32)
    for off in (64, 80, 96, 112):
        idx_v[NCHUNK - 1, pl.ds(off, 16)] = zeros16

    def group_body(t, carry):
        r0 = (wid * GROUPS_PER_W + t) * LPG
        pltpu.sync_copy(rb.at[pl.ds(r0, LPG)], cb0)
        pltpu.sync_copy(rx1.at[pl.ds(r0, LPG)], cb1)
        pltpu.sync_copy(ry1.at[pl.ds(r0, LPG)], cb2)
        pltpu.sync_copy(rx2.at[pl.ds(r0, LPG)], cb3)
        pltpu.sync_copy(ry2.at[pl.ds(r0, LPG)], cb4)

        bint = cb0[...].astype(jnp.int32)
        rsw = (cb1[...] + 0.5).astype(jnp.int32).astype(jnp.float32) * SCALE
        rsh = (cb2[...] + 0.5).astype(jnp.int32).astype(jnp.float32) * SCALE
        rew = (cb3[...] + 1.5).astype(jnp.int32).astype(jnp.float32) * SCALE
        reh = (cb4[...] + 1.5).astype(jnp.int32).astype(jnp.float32) * SCALE
        roi_w = jnp.maximum(rew - rsw, 0.1)
        roi_h = jnp.maximum(reh - rsh, 0.1)
        bsw = roi_w / 7.0
        bsh = roi_h / 7.0

        hs_l, he_l, ws_l, we_l = [], [], [], []
        for p in range(G):
            vs = jnp.float32(p) * bsh + rsh
            hs_l.append(jnp.clip(vs.astype(jnp.int32), 0, H))
            ve = jnp.float32(p + 1) * bsh + rsh
            te = ve.astype(jnp.int32)
            te = jnp.where(te.astype(jnp.float32) < ve, te + 1, te)
            he_l.append(jnp.clip(te, 0, H))
            us = jnp.float32(p) * bsw + rsw
            ws_l.append(jnp.clip(us.astype(jnp.int32), 0, W))
            ue = jnp.float32(p + 1) * bsw + rsw
            tu = ue.astype(jnp.int32)
            tu = jnp.where(tu.astype(jnp.float32) < ue, tu + 1, tu)
            we_l.append(jnp.clip(tu, 0, W))

        for ph in range(G):
            for pw in range(G):
                k = ph * G + pw
                hs, he = hs_l[ph], he_l[ph]
                ws, we = ws_l[pw], we_l[pw]
                base = (bint * NB + k) * (HP * HP)
                q = k * 64
                j, off = q // 128, q % 128
                idx_v[j, pl.ds(off, 16)] = base + he * HP + we
                idx_v[j, pl.ds(off + 16, 16)] = base + hs * HP + we
                idx_v[j, pl.ds(off + 32, 16)] = base + he * HP + ws
                idx_v[j, pl.ds(off + 48, 16)] = base + hs * HP + ws
                area = ((he - hs) * (we - ws)).astype(jnp.float32)
                empty = (he <= hs) | (we <= ws)
                fac_v[k, :] = jnp.where(empty, 0.0,
                                        1.0 / jnp.maximum(area, 1.0))

        copies = [
            pltpu.async_copy(table.at[idx_v.at[j]],
                             rows_v.at[pl.ds(j * 128, 128)], sem)
            for j in range(NCHUNK)
        ]
        for cp in copies:
            cp.wait()

        def roi_body(l, c2):
            lvec = jnp.full((16,), l, jnp.int32)
            for k in range(NB):
                ree = k * 64 + l
                s0 = (rows_v[ree, pl.ds(0, 16)]
                      - rows_v[ree + 16, pl.ds(0, 16)]
                      - rows_v[ree + 32, pl.ds(0, 16)]
                      + rows_v[ree + 48, pl.ds(0, 16)])
                s1 = (rows_v[ree, pl.ds(16, 16)]
                      - rows_v[ree + 16, pl.ds(16, 16)]
                      - rows_v[ree + 32, pl.ds(16, 16)]
                      + rows_v[ree + 48, pl.ds(16, 16)])
                f = fac_v[k, :].at[lvec].get(mode="promise_in_bounds")
                outb[k, pl.ds(0, 16)] = s0 * f
                outb[k, pl.ds(16, 16)] = s1 * f
            pltpu.sync_copy(outb, out.at[r0 + l])
            return c2

        lax.fori_loop(0, LPG, roi_body, 0)
        return carry

    lax.fori_loop(0, GROUPS_PER_W, group_body, 0)


def kernel(features, rois):
    B, C, _, _ = features.shape
    feats2 = (features.reshape(B, D, NB, H, W)
              .transpose(0, 2, 1, 3, 4)
              .reshape(NG, D, H, W))
    cs = _integral(feats2)
    ii = jnp.pad(cs, ((0, 0), (1, 0), (0, 0), (1, 0)))
    table = jnp.pad(ii.transpose(0, 1, 3, 2),
                    ((0, 0), (0, 0), (0, 0), (0, DP - D)))
    table = table.reshape(NG * HP * HP, DP)

    roisp = jnp.pad(rois, ((0, NPAD - NROI), (0, 0)))
    out_sc = _sc_pool(table, roisp[:, 0], roisp[:, 1], roisp[:, 2],
                      roisp[:, 3], roisp[:, 4])
    return (out_sc[:NROI, :, :D]
            .transpose(0, 2, 1)
            .reshape(NROI, D, G, G))
